# baseline (speedup 1.0000x reference)
import functools

import jax
import jax.numpy as jnp
from jax import lax
from jax.experimental import pallas as pl
from jax.experimental.pallas import tpu as pltpu
from jax.experimental.pallas import tpu_sc as plsc

_K_QQ = 16
_K_QC = 64
_RB = 256
_HI = jax.lax.Precision.HIGHEST


def _monotone_key(x_f32):
    s = jax.lax.bitcast_convert_type(x_f32, jnp.int32)
    return s ^ (jax.lax.shift_right_arithmetic(s, 31) & jnp.int32(0x7FFFFFFF))


def _key_to_float(k_i32):
    s = k_i32 ^ (jax.lax.shift_right_arithmetic(k_i32, 31) & jnp.int32(0x7FFFFFFF))
    return jax.lax.bitcast_convert_type(s, jnp.float32)


def _dist_kernel(q_ref, b_ref, d2_ref):
    q = q_ref[...]
    b = b_ref[...]
    q2 = jnp.sum(q * q, axis=1, keepdims=True)
    b2 = jnp.sum(b * b, axis=1)
    qb = jax.lax.dot_general(q, b, (((1,), (1,)), ((), ())),
                             preferred_element_type=jnp.float32, precision=_HI)
    d2_ref[...] = q2 + b2[None, :] - 2.0 * qb


def _dist(qc8, bc8):
    n = qc8.shape[0]
    m = bc8.shape[0]
    return pl.pallas_call(
        _dist_kernel,
        grid=(n // _RB,),
        in_specs=[
            pl.BlockSpec((_RB, 8), lambda i: (i, 0)),
            pl.BlockSpec((m, 8), lambda i: (0, 0)),
        ],
        out_specs=pl.BlockSpec((_RB, m), lambda i: (i, 0)),
        out_shape=jax.ShapeDtypeStruct((n, m), jnp.float32),
    )(qc8, bc8)


def _sc_select(d2, kk):
    n_rows, n_cols = d2.shape
    info = plsc.get_sparse_core_info()
    nc, ns, L = info.num_cores, info.num_subcores, info.num_lanes
    nw = nc * ns
    rpw = n_rows // nw
    groups = rpw // L
    chunk = 2048
    nchunk = n_cols // chunk
    mesh = plsc.VectorSubcoreMesh(core_axis_name="c", subcore_axis_name="s")

    @functools.partial(
        pl.kernel, mesh=mesh,
        compiler_params=pltpu.CompilerParams(needs_layout_passes=False),
        out_type=jax.ShapeDtypeStruct((n_rows,), jnp.float32),
        scratch_types=[
            pltpu.VMEM((L * chunk,), jnp.float32),
            pltpu.VMEM((256 * L,), jnp.int32),
            pltpu.VMEM((L * n_cols,), jnp.int32),
            pltpu.VMEM((L,), jnp.float32),
            pltpu.SemaphoreType.DMA,
        ],
    )
    def sel(d2_hbm, out_hbm, buf, hist, cand, tout, sem):
        wid = lax.axis_index("s") * nc + lax.axis_index("c")
        lane = lax.iota(jnp.int32, L)
        zero = jnp.zeros((L,), jnp.int32)
        one = jnp.ones((L,), jnp.int32)
        lane_base = lane * chunk

        def load_chunk(base, ci):
            handles = []
            for l in range(L):
                src = d2_hbm.at[pl.ds((base + l) * n_cols + ci * chunk, chunk)]
                dst = buf.at[pl.ds(l * chunk, chunk)]
                handles.append(pltpu.async_copy(src, dst, sem))
            for h in handles:
                h.wait()

        def group_body(g, _):
            base = wid * rpw + g * L

            def zero_hist(b, _):
                hist[pl.ds(b * L, L)] = zero
                return 0
            lax.fori_loop(0, 256, zero_hist, 0)

            def hchunk(ci, _):
                load_chunk(base, ci)

                def hcol(cc, _):
                    for u in range(4):
                        idx = lane_base + cc * 4 + u
                        v = plsc.load_gather(buf, [idx])
                        raw = plsc.bitcast(v, jnp.int32)
                        digit = (raw >> 24) & 0xFF
                        plsc.addupdate_scatter(
                            hist, [(digit << 4) + lane], one)
                    return 0
                lax.fori_loop(0, chunk // 4, hcol, 0)
                return 0
            lax.fori_loop(0, nchunk, hchunk, 0)

            def bscan1(t, carry):
                acc, dig, basec = carry
                b = jnp.where(t < 128, 255 - t, t - 128)
                h = hist[pl.ds(b * L, L)]
                acc2 = acc + h
                take = (acc < kk) & (acc2 >= kk)
                dig = jnp.where(take, b, dig)
                basec = jnp.where(take, acc, basec)
                return acc2, dig, basec
            _, dig1, base1 = lax.fori_loop(0, 256, bscan1,
                                           (zero, zero, zero))
            r = kk - base1
            inv24 = jnp.where(dig1 >= 128, jnp.int32(0x00FFFFFF), zero)

            def cchunk(ci, cnt):
                load_chunk(base, ci)

                def ccol(cc, cnt):
                    for u in range(4):
                        idx = lane_base + cc * 4 + u
                        v = plsc.load_gather(buf, [idx])
                        raw = plsc.bitcast(v, jnp.int32)
                        m = ((raw >> 24) & 0xFF) == dig1
                        val = (raw & 0x00FFFFFF) ^ inv24
                        cidx = lane * n_cols + cnt
                        plsc.store_scatter(cand, [cidx], val, mask=m)
                        cnt = cnt + jnp.where(m, one, zero)
                    return cnt
                return lax.fori_loop(0, chunk // 4, ccol, cnt)
            cnt = lax.fori_loop(0, nchunk, cchunk, zero)

            pref = zero
            for shift in (16, 8, 0):
                lax.fori_loop(0, 256, zero_hist, 0)
                maxcnt = jnp.max(cnt)

                def hscan(j, _):
                    m = j < cnt
                    v = plsc.load_gather(cand, [lane * n_cols + j], mask=m)
                    digit = (v >> shift) & 0xFF
                    plsc.addupdate_scatter(hist, [(digit << 4) + lane], one,
                                           mask=m)
                    return 0
                lax.fori_loop(0, maxcnt, hscan, 0)

                def bscan2(b, carry):
                    acc, dig, basec = carry
                    h = hist[pl.ds(b * L, L)]
                    acc2 = acc + h
                    take = (acc < r) & (acc2 >= r)
                    dig = jnp.where(take, b, dig)
                    basec = jnp.where(take, acc, basec)
                    return acc2, dig, basec
                _, dig, basec = lax.fori_loop(0, 256, bscan2,
                                              (zero, zero, zero))
                r = r - basec
                pref = pref | (dig << shift)

                if shift != 0:
                    def cscan(j, c2):
                        m = j < cnt
                        v = plsc.load_gather(cand, [lane * n_cols + j],
                                             mask=m)
                        keep = m & (((v >> shift) & 0xFF) == dig)
                        plsc.store_scatter(cand, [lane * n_cols + c2], v,
                                           mask=keep)
                        return c2 + jnp.where(keep, one, zero)
                    cnt = lax.fori_loop(0, maxcnt, cscan, zero)

            raw_t = (dig1 << 24) | (pref ^ inv24)
            tout[...] = plsc.bitcast(raw_t, jnp.float32)
            pltpu.sync_copy(tout, out_hbm.at[pl.ds(base, L)])
            return 0

        lax.fori_loop(0, groups, group_body, 0)

    return sel(d2.reshape(-1))


def _split_kernel(x_ref, hi_ref, lo_ref):
    x = x_ref[...]
    hi = x.astype(jnp.bfloat16)
    hi_ref[...] = hi
    lo_ref[...] = (x - hi.astype(jnp.float32)).astype(jnp.bfloat16)


def _split(x):
    n, d = x.shape
    return pl.pallas_call(
        _split_kernel,
        grid=(n // _RB,),
        in_specs=[pl.BlockSpec((_RB, d), lambda i: (i, 0))],
        out_specs=[
            pl.BlockSpec((_RB, d), lambda i: (i, 0)),
            pl.BlockSpec((_RB, d), lambda i: (i, 0)),
        ],
        out_shape=[
            jax.ShapeDtypeStruct((n, d), jnp.bfloat16),
            jax.ShapeDtypeStruct((n, d), jnp.bfloat16),
        ],
    )(x)


def _attn_kernel(d2_ref, t_ref, gfhi_ref, gflo_ref, sf_ref, w_ref, out_ref):
    d2 = d2_ref[...]
    mask = jnp.where(d2 <= t_ref[...], 1.0, 0.0)
    maskb = mask.astype(jnp.bfloat16)
    ones = jnp.ones((d2.shape[1], 8), jnp.bfloat16)
    dn = (((1,), (0,)), ((), ()))
    cnt = jax.lax.dot_general(maskb, ones, dn,
                              preferred_element_type=jnp.float32)[:, :1]
    acc = (jax.lax.dot_general(maskb, gfhi_ref[...], dn,
                               preferred_element_type=jnp.float32)
           + jax.lax.dot_general(maskb, gflo_ref[...], dn,
                                 preferred_element_type=jnp.float32))
    mean = acc / cnt
    up = jax.lax.dot_general(mean, w_ref[...], dn,
                             preferred_element_type=jnp.float32, precision=_HI)
    out_ref[...] = sf_ref[...] + up


def _attn_step(d2, t, gfeat_hilo, sfeat, w):
    gfhi, gflo = gfeat_hilo
    n, m = d2.shape
    d = gfhi.shape[1]
    return pl.pallas_call(
        _attn_kernel,
        grid=(n // _RB,),
        in_specs=[
            pl.BlockSpec((_RB, m), lambda i: (i, 0)),
            pl.BlockSpec((_RB, 1), lambda i: (i, 0)),
            pl.BlockSpec((m, d), lambda i: (0, 0)),
            pl.BlockSpec((m, d), lambda i: (0, 0)),
            pl.BlockSpec((_RB, d), lambda i: (i, 0)),
            pl.BlockSpec((d, d), lambda i: (0, 0)),
        ],
        out_specs=pl.BlockSpec((_RB, d), lambda i: (i, 0)),
        out_shape=jax.ShapeDtypeStruct((n, d), jnp.float32),
    )(d2, t, gfhi, gflo, sfeat, w)


def _sim_stats_kernel(tfhi_ref, tflo_ref, afhi_ref, aflo_ref, sim_ref,
                      rmax_ref, rsum_ref, cmax_ref, csum_ref):
    d = tfhi_ref.shape[1]
    dn = (((1,), (1,)), ((), ()))
    tfhi = tfhi_ref[...]
    afhi = afhi_ref[...]
    s = (jax.lax.dot_general(tfhi, afhi, dn,
                             preferred_element_type=jnp.float32)
         + jax.lax.dot_general(tfhi, aflo_ref[...], dn,
                               preferred_element_type=jnp.float32)
         + jax.lax.dot_general(tflo_ref[...], afhi, dn,
                               preferred_element_type=jnp.float32))
    s = s * (1.0 / jnp.sqrt(jnp.float32(d)))
    sim_ref[...] = s
    rmax = jnp.max(s, axis=1, keepdims=True)
    rmax_ref[...] = rmax
    e = jnp.exp(s - rmax)
    ones = jnp.ones((s.shape[1], 8), jnp.float32)
    rsum_ref[...] = jax.lax.dot_general(e, ones, (((1,), (0,)), ((), ())),
                                        preferred_element_type=jnp.float32)[:, :1]
    cmax = jnp.max(s, axis=0)
    cmax_ref[0, 0, :] = cmax
    csum_ref[0, 0, :] = jnp.sum(jnp.exp(s - cmax[None, :]), axis=0)


def _sim_stats(tf_hilo, af_hilo):
    tfhi, tflo = tf_hilo
    afhi, aflo = af_hilo
    n, d = tfhi.shape
    m = afhi.shape[0]
    g = n // _RB
    return pl.pallas_call(
        _sim_stats_kernel,
        grid=(g,),
        in_specs=[
            pl.BlockSpec((_RB, d), lambda i: (i, 0)),
            pl.BlockSpec((_RB, d), lambda i: (i, 0)),
            pl.BlockSpec((m, d), lambda i: (0, 0)),
            pl.BlockSpec((m, d), lambda i: (0, 0)),
        ],
        out_specs=[
            pl.BlockSpec((_RB, m), lambda i: (i, 0)),
            pl.BlockSpec((_RB, 1), lambda i: (i, 0)),
            pl.BlockSpec((_RB, 1), lambda i: (i, 0)),
            pl.BlockSpec((1, 1, m), lambda i: (i, 0, 0)),
            pl.BlockSpec((1, 1, m), lambda i: (i, 0, 0)),
        ],
        out_shape=[
            jax.ShapeDtypeStruct((n, m), jnp.float32),
            jax.ShapeDtypeStruct((n, 1), jnp.float32),
            jax.ShapeDtypeStruct((n, 1), jnp.float32),
            jax.ShapeDtypeStruct((g, 1, m), jnp.float32),
            jax.ShapeDtypeStruct((g, 1, m), jnp.float32),
        ],
    )(tfhi, tflo, afhi, aflo)


def _dual_softmax_kernel(sim_ref, rmax_ref, rsum_ref, cmaxp_ref, csump_ref,
                         out_ref):
    s = sim_ref[...]
    g = cmaxp_ref.shape[0]
    m = cmaxp_ref.shape[2]
    cmaxp = cmaxp_ref[...].reshape(g, m)
    csump = csump_ref[...].reshape(g, m)
    cmax = jnp.max(cmaxp, axis=0)
    csum = jnp.sum(csump * jnp.exp(cmaxp - cmax[None, :]), axis=0)
    num = jnp.exp((s - rmax_ref[...]) + (s - cmax[None, :]))
    out_ref[...] = num / (rsum_ref[...] * csum[None, :])


def _dual_softmax(sim, rmax, rsum, cmaxp, csump):
    n, m = sim.shape
    g = cmaxp.shape[0]
    return pl.pallas_call(
        _dual_softmax_kernel,
        grid=(n // _RB,),
        in_specs=[
            pl.BlockSpec((_RB, m), lambda i: (i, 0)),
            pl.BlockSpec((_RB, 1), lambda i: (i, 0)),
            pl.BlockSpec((_RB, 1), lambda i: (i, 0)),
            pl.BlockSpec((g, 1, m), lambda i: (0, 0, 0)),
            pl.BlockSpec((g, 1, m), lambda i: (0, 0, 0)),
        ],
        out_specs=pl.BlockSpec((_RB, m), lambda i: (i, 0)),
        out_shape=jax.ShapeDtypeStruct((n, m), jnp.float32),
    )(sim, rmax, rsum, cmaxp, csump)


def kernel(anchor_coord, anchor_feat, anchor_offset, target_coord,
           target_feat, target_offset, Wq1, Wc1, Wq2, Wc2):
    tc8 = jnp.pad(target_coord, ((0, 0), (0, 5)))
    ac8 = jnp.pad(anchor_coord, ((0, 0), (0, 5)))

    dtt = _dist(tc8, tc8)
    dta = _dist(tc8, ac8)
    dat = _dist(ac8, tc8)
    t_tt = _sc_select(dtt, _K_QQ).reshape(-1, 1)
    t_ta = _sc_select(dta, _K_QC).reshape(-1, 1)
    t_at = _sc_select(dat, _K_QC).reshape(-1, 1)

    tf = target_feat
    af = anchor_feat
    tf_hl = _split(tf)
    af_hl = _split(af)
    for (wq, wc) in ((Wq1, Wc1), (Wq2, Wc2)):
        tf = _attn_step(dtt, t_tt, tf_hl, tf, wq)
        tf = _attn_step(dta, t_ta, af_hl, tf, wc)
        tf_hl = _split(tf)
        af = _attn_step(dat, t_at, tf_hl, af, wc)
        af_hl = _split(af)

    sim, rmax, rsum, cmaxp, csump = _sim_stats(tf_hl, af_hl)
    return _dual_softmax(sim, rmax, rsum, cmaxp, csump)


# speedup vs baseline: 1.9189x; 1.9189x over previous
import functools

import jax
import jax.numpy as jnp
from jax import lax
from jax.experimental import pallas as pl
from jax.experimental.pallas import tpu as pltpu
from jax.experimental.pallas import tpu_sc as plsc

_K_QQ = 16
_K_QC = 64
_RB = 256
_HI = jax.lax.Precision.HIGHEST


def _monotone_key(x_f32):
    s = jax.lax.bitcast_convert_type(x_f32, jnp.int32)
    return s ^ (jax.lax.shift_right_arithmetic(s, 31) & jnp.int32(0x7FFFFFFF))


def _key_to_float(k_i32):
    s = k_i32 ^ (jax.lax.shift_right_arithmetic(k_i32, 31) & jnp.int32(0x7FFFFFFF))
    return jax.lax.bitcast_convert_type(s, jnp.float32)


def _dist_kernel(q_ref, b_ref, d2_ref):
    q = q_ref[...]
    b = b_ref[...]
    q2 = jnp.sum(q * q, axis=1, keepdims=True)
    b2 = jnp.sum(b * b, axis=1)
    qb = jax.lax.dot_general(q, b, (((1,), (1,)), ((), ())),
                             preferred_element_type=jnp.float32, precision=_HI)
    d2_ref[...] = q2 + b2[None, :] - 2.0 * qb


def _dist(qc8, bc8):
    n = qc8.shape[0]
    m = bc8.shape[0]
    return pl.pallas_call(
        _dist_kernel,
        grid=(n // _RB,),
        in_specs=[
            pl.BlockSpec((_RB, 8), lambda i: (i, 0)),
            pl.BlockSpec((m, 8), lambda i: (0, 0)),
        ],
        out_specs=pl.BlockSpec((_RB, m), lambda i: (i, 0)),
        out_shape=jax.ShapeDtypeStruct((n, m), jnp.float32),
    )(qc8, bc8)


def _sc_select(d2, kk):
    n_rows, n_cols = d2.shape
    info = plsc.get_sparse_core_info()
    nc, ns, L = info.num_cores, info.num_subcores, info.num_lanes
    nw = nc * ns
    rpw = n_rows // nw
    groups = rpw // L
    chunk = 2048
    nchunk = n_cols // chunk
    mesh = plsc.VectorSubcoreMesh(core_axis_name="c", subcore_axis_name="s")

    @functools.partial(
        pl.kernel, mesh=mesh,
        compiler_params=pltpu.CompilerParams(needs_layout_passes=False),
        out_type=jax.ShapeDtypeStruct((n_rows,), jnp.float32),
        scratch_types=[
            pltpu.VMEM((L * chunk,), jnp.float32),
            pltpu.VMEM((256 * L,), jnp.int32),
            pltpu.VMEM((L * n_cols,), jnp.int32),
            pltpu.VMEM((L,), jnp.float32),
            pltpu.SemaphoreType.DMA,
        ],
    )
    def sel(d2_hbm, out_hbm, buf, hist, cand, tout, sem):
        wid = lax.axis_index("s") * nc + lax.axis_index("c")
        lane = lax.iota(jnp.int32, L)
        zero = jnp.zeros((L,), jnp.int32)
        one = jnp.ones((L,), jnp.int32)
        lane_base = lane * chunk

        def load_chunk(base, ci):
            handles = []
            for l in range(L):
                src = d2_hbm.at[pl.ds((base + l) * n_cols + ci * chunk, chunk)]
                dst = buf.at[pl.ds(l * chunk, chunk)]
                handles.append(pltpu.async_copy(src, dst, sem))
            for h in handles:
                h.wait()

        def group_body(g, _):
            base = wid * rpw + g * L

            @plsc.parallel_loop(0, 256, unroll=8)
            def _(b):
                hist[pl.ds(b * L, L)] = zero

            def hchunk(ci, _):
                load_chunk(base, ci)

                @plsc.parallel_loop(0, chunk, unroll=8)
                def _(cc):
                    v = plsc.load_gather(buf, [lane_base + cc])
                    raw = plsc.bitcast(v, jnp.int32)
                    digit = (raw >> 24) & 0xFF
                    plsc.addupdate_scatter(hist, [(digit << 4) + lane], one)
                return 0
            lax.fori_loop(0, nchunk, hchunk, 0)

            def bscan1(t, carry):
                acc, dig, basec = carry
                b = jnp.where(t < 128, 255 - t, t - 128)
                h = hist[pl.ds(b * L, L)]
                acc2 = acc + h
                take = (acc < kk) & (acc2 >= kk)
                dig = jnp.where(take, b, dig)
                basec = jnp.where(take, acc, basec)
                return acc2, dig, basec
            _, dig1, base1 = plsc.parallel_loop(
                0, 256, unroll=4, carry=(zero, zero, zero))(bscan1)
            r = kk - base1
            inv24 = jnp.where(dig1 >= 128, jnp.int32(0x00FFFFFF), zero)

            def cchunk(ci, cnt):
                load_chunk(base, ci)

                def ccol(cc, cnt):
                    v = plsc.load_gather(buf, [lane_base + cc])
                    raw = plsc.bitcast(v, jnp.int32)
                    m = ((raw >> 24) & 0xFF) == dig1
                    val = (raw & 0x00FFFFFF) ^ inv24
                    cidx = lane * n_cols + cnt
                    plsc.store_scatter(cand, [cidx], val, mask=m)
                    return cnt + jnp.where(m, one, zero)
                return plsc.parallel_loop(0, chunk, unroll=4,
                                          carry=cnt)(ccol)
            cnt = lax.fori_loop(0, nchunk, cchunk, zero)

            pref = zero
            for shift in (16, 8, 0):
                @plsc.parallel_loop(0, 256, unroll=8)
                def _(b):
                    hist[pl.ds(b * L, L)] = zero
                maxcnt = jnp.max(cnt)

                def hscan(j):
                    m = j < cnt
                    v = plsc.load_gather(cand, [lane * n_cols + j], mask=m)
                    digit = (v >> shift) & 0xFF
                    plsc.addupdate_scatter(hist, [(digit << 4) + lane], one,
                                           mask=m)
                plsc.parallel_loop(0, maxcnt, unroll=4)(hscan)

                def bscan2(b, carry):
                    acc, dig, basec = carry
                    h = hist[pl.ds(b * L, L)]
                    acc2 = acc + h
                    take = (acc < r) & (acc2 >= r)
                    dig = jnp.where(take, b, dig)
                    basec = jnp.where(take, acc, basec)
                    return acc2, dig, basec
                _, dig, basec = plsc.parallel_loop(
                    0, 256, unroll=4, carry=(zero, zero, zero))(bscan2)
                r = r - basec
                pref = pref | (dig << shift)

                if shift != 0:
                    def cscan(j, c2):
                        m = j < cnt
                        v = plsc.load_gather(cand, [lane * n_cols + j],
                                             mask=m)
                        keep = m & (((v >> shift) & 0xFF) == dig)
                        plsc.store_scatter(cand, [lane * n_cols + c2], v,
                                           mask=keep)
                        return c2 + jnp.where(keep, one, zero)
                    cnt = plsc.parallel_loop(0, maxcnt, unroll=4,
                                             carry=zero)(cscan)

            raw_t = (dig1 << 24) | (pref ^ inv24)
            tout[...] = plsc.bitcast(raw_t, jnp.float32)
            pltpu.sync_copy(tout, out_hbm.at[pl.ds(base, L)])
            return 0

        lax.fori_loop(0, groups, group_body, 0)

    return sel(d2.reshape(-1))


def _split_kernel(x_ref, hi_ref, lo_ref):
    x = x_ref[...]
    hi = x.astype(jnp.bfloat16)
    hi_ref[...] = hi
    lo_ref[...] = (x - hi.astype(jnp.float32)).astype(jnp.bfloat16)


def _split(x):
    n, d = x.shape
    return pl.pallas_call(
        _split_kernel,
        grid=(n // _RB,),
        in_specs=[pl.BlockSpec((_RB, d), lambda i: (i, 0))],
        out_specs=[
            pl.BlockSpec((_RB, d), lambda i: (i, 0)),
            pl.BlockSpec((_RB, d), lambda i: (i, 0)),
        ],
        out_shape=[
            jax.ShapeDtypeStruct((n, d), jnp.bfloat16),
            jax.ShapeDtypeStruct((n, d), jnp.bfloat16),
        ],
    )(x)


def _attn_kernel(d2_ref, t_ref, gfhi_ref, gflo_ref, sf_ref, w_ref, out_ref):
    d2 = d2_ref[...]
    mask = jnp.where(d2 <= t_ref[...], 1.0, 0.0)
    maskb = mask.astype(jnp.bfloat16)
    ones = jnp.ones((d2.shape[1], 8), jnp.bfloat16)
    dn = (((1,), (0,)), ((), ()))
    cnt = jax.lax.dot_general(maskb, ones, dn,
                              preferred_element_type=jnp.float32)[:, :1]
    acc = (jax.lax.dot_general(maskb, gfhi_ref[...], dn,
                               preferred_element_type=jnp.float32)
           + jax.lax.dot_general(maskb, gflo_ref[...], dn,
                                 preferred_element_type=jnp.float32))
    mean = acc / cnt
    up = jax.lax.dot_general(mean, w_ref[...], dn,
                             preferred_element_type=jnp.float32, precision=_HI)
    out_ref[...] = sf_ref[...] + up


def _attn_step(d2, t, gfeat_hilo, sfeat, w):
    gfhi, gflo = gfeat_hilo
    n, m = d2.shape
    d = gfhi.shape[1]
    return pl.pallas_call(
        _attn_kernel,
        grid=(n // _RB,),
        in_specs=[
            pl.BlockSpec((_RB, m), lambda i: (i, 0)),
            pl.BlockSpec((_RB, 1), lambda i: (i, 0)),
            pl.BlockSpec((m, d), lambda i: (0, 0)),
            pl.BlockSpec((m, d), lambda i: (0, 0)),
            pl.BlockSpec((_RB, d), lambda i: (i, 0)),
            pl.BlockSpec((d, d), lambda i: (0, 0)),
        ],
        out_specs=pl.BlockSpec((_RB, d), lambda i: (i, 0)),
        out_shape=jax.ShapeDtypeStruct((n, d), jnp.float32),
    )(d2, t, gfhi, gflo, sfeat, w)


def _sim_stats_kernel(tfhi_ref, tflo_ref, afhi_ref, aflo_ref, sim_ref,
                      rmax_ref, rsum_ref, cmax_ref, csum_ref):
    d = tfhi_ref.shape[1]
    dn = (((1,), (1,)), ((), ()))
    tfhi = tfhi_ref[...]
    afhi = afhi_ref[...]
    s = (jax.lax.dot_general(tfhi, afhi, dn,
                             preferred_element_type=jnp.float32)
         + jax.lax.dot_general(tfhi, aflo_ref[...], dn,
                               preferred_element_type=jnp.float32)
         + jax.lax.dot_general(tflo_ref[...], afhi, dn,
                               preferred_element_type=jnp.float32))
    s = s * (1.0 / jnp.sqrt(jnp.float32(d)))
    sim_ref[...] = s
    rmax = jnp.max(s, axis=1, keepdims=True)
    rmax_ref[...] = rmax
    e = jnp.exp(s - rmax)
    ones = jnp.ones((s.shape[1], 8), jnp.float32)
    rsum_ref[...] = jax.lax.dot_general(e, ones, (((1,), (0,)), ((), ())),
                                        preferred_element_type=jnp.float32)[:, :1]
    cmax = jnp.max(s, axis=0)
    cmax_ref[0, 0, :] = cmax
    csum_ref[0, 0, :] = jnp.sum(jnp.exp(s - cmax[None, :]), axis=0)


def _sim_stats(tf_hilo, af_hilo):
    tfhi, tflo = tf_hilo
    afhi, aflo = af_hilo
    n, d = tfhi.shape
    m = afhi.shape[0]
    g = n // _RB
    return pl.pallas_call(
        _sim_stats_kernel,
        grid=(g,),
        in_specs=[
            pl.BlockSpec((_RB, d), lambda i: (i, 0)),
            pl.BlockSpec((_RB, d), lambda i: (i, 0)),
            pl.BlockSpec((m, d), lambda i: (0, 0)),
            pl.BlockSpec((m, d), lambda i: (0, 0)),
        ],
        out_specs=[
            pl.BlockSpec((_RB, m), lambda i: (i, 0)),
            pl.BlockSpec((_RB, 1), lambda i: (i, 0)),
            pl.BlockSpec((_RB, 1), lambda i: (i, 0)),
            pl.BlockSpec((1, 1, m), lambda i: (i, 0, 0)),
            pl.BlockSpec((1, 1, m), lambda i: (i, 0, 0)),
        ],
        out_shape=[
            jax.ShapeDtypeStruct((n, m), jnp.float32),
            jax.ShapeDtypeStruct((n, 1), jnp.float32),
            jax.ShapeDtypeStruct((n, 1), jnp.float32),
            jax.ShapeDtypeStruct((g, 1, m), jnp.float32),
            jax.ShapeDtypeStruct((g, 1, m), jnp.float32),
        ],
    )(tfhi, tflo, afhi, aflo)


def _dual_softmax_kernel(sim_ref, rmax_ref, rsum_ref, cmaxp_ref, csump_ref,
                         out_ref):
    s = sim_ref[...]
    g = cmaxp_ref.shape[0]
    m = cmaxp_ref.shape[2]
    cmaxp = cmaxp_ref[...].reshape(g, m)
    csump = csump_ref[...].reshape(g, m)
    cmax = jnp.max(cmaxp, axis=0)
    csum = jnp.sum(csump * jnp.exp(cmaxp - cmax[None, :]), axis=0)
    num = jnp.exp((s - rmax_ref[...]) + (s - cmax[None, :]))
    out_ref[...] = num / (rsum_ref[...] * csum[None, :])


def _dual_softmax(sim, rmax, rsum, cmaxp, csump):
    n, m = sim.shape
    g = cmaxp.shape[0]
    return pl.pallas_call(
        _dual_softmax_kernel,
        grid=(n // _RB,),
        in_specs=[
            pl.BlockSpec((_RB, m), lambda i: (i, 0)),
            pl.BlockSpec((_RB, 1), lambda i: (i, 0)),
            pl.BlockSpec((_RB, 1), lambda i: (i, 0)),
            pl.BlockSpec((g, 1, m), lambda i: (0, 0, 0)),
            pl.BlockSpec((g, 1, m), lambda i: (0, 0, 0)),
        ],
        out_specs=pl.BlockSpec((_RB, m), lambda i: (i, 0)),
        out_shape=jax.ShapeDtypeStruct((n, m), jnp.float32),
    )(sim, rmax, rsum, cmaxp, csump)


def kernel(anchor_coord, anchor_feat, anchor_offset, target_coord,
           target_feat, target_offset, Wq1, Wc1, Wq2, Wc2):
    tc8 = jnp.pad(target_coord, ((0, 0), (0, 5)))
    ac8 = jnp.pad(anchor_coord, ((0, 0), (0, 5)))

    dtt = _dist(tc8, tc8)
    dta = _dist(tc8, ac8)
    dat = _dist(ac8, tc8)
    t_tt = _sc_select(dtt, _K_QQ).reshape(-1, 1)
    t_ta = _sc_select(dta, _K_QC).reshape(-1, 1)
    t_at = _sc_select(dat, _K_QC).reshape(-1, 1)

    tf = target_feat
    af = anchor_feat
    tf_hl = _split(tf)
    af_hl = _split(af)
    for (wq, wc) in ((Wq1, Wc1), (Wq2, Wc2)):
        tf = _attn_step(dtt, t_tt, tf_hl, tf, wq)
        tf = _attn_step(dta, t_ta, af_hl, tf, wc)
        tf_hl = _split(tf)
        af = _attn_step(dat, t_at, tf_hl, af, wc)
        af_hl = _split(af)

    sim, rmax, rsum, cmaxp, csump = _sim_stats(tf_hl, af_hl)
    return _dual_softmax(sim, rmax, rsum, cmaxp, csump)


# speedup vs baseline: 3.7248x; 1.9411x over previous
import functools

import jax
import jax.numpy as jnp
from jax import lax
from jax.experimental import pallas as pl
from jax.experimental.pallas import tpu as pltpu
from jax.experimental.pallas import tpu_sc as plsc

_K_QQ = 16
_K_QC = 64
_RB = 256
_HI = jax.lax.Precision.HIGHEST


def _monotone_key(x_f32):
    s = jax.lax.bitcast_convert_type(x_f32, jnp.int32)
    return s ^ (jax.lax.shift_right_arithmetic(s, 31) & jnp.int32(0x7FFFFFFF))


def _key_to_float(k_i32):
    s = k_i32 ^ (jax.lax.shift_right_arithmetic(k_i32, 31) & jnp.int32(0x7FFFFFFF))
    return jax.lax.bitcast_convert_type(s, jnp.float32)


def _dist_kernel(q_ref, b_ref, d2_ref):
    q = q_ref[...]
    b = b_ref[...]
    q2 = jnp.sum(q * q, axis=1, keepdims=True)
    b2 = jnp.sum(b * b, axis=1)
    qb = jax.lax.dot_general(q, b, (((1,), (1,)), ((), ())),
                             preferred_element_type=jnp.float32, precision=_HI)
    d2_ref[...] = q2 + b2[None, :] - 2.0 * qb


def _dist(qc8, bc8):
    n = qc8.shape[0]
    m = bc8.shape[0]
    return pl.pallas_call(
        _dist_kernel,
        grid=(n // _RB,),
        in_specs=[
            pl.BlockSpec((_RB, 8), lambda i: (i, 0)),
            pl.BlockSpec((m, 8), lambda i: (0, 0)),
        ],
        out_specs=pl.BlockSpec((_RB, m), lambda i: (i, 0)),
        out_shape=jax.ShapeDtypeStruct((n, m), jnp.float32),
    )(qc8, bc8)


def _sc_select(d2, kk):
    n_rows, n_cols = d2.shape
    info = plsc.get_sparse_core_info()
    nc, ns, L = info.num_cores, info.num_subcores, info.num_lanes
    nw = nc * ns
    rpw = n_rows // nw
    groups = rpw // L
    chunk = 2048
    nchunk = n_cols // chunk
    mesh = plsc.VectorSubcoreMesh(core_axis_name="c", subcore_axis_name="s")

    @functools.partial(
        pl.kernel, mesh=mesh,
        compiler_params=pltpu.CompilerParams(needs_layout_passes=False),
        out_type=jax.ShapeDtypeStruct((n_rows,), jnp.float32),
        scratch_types=[
            pltpu.VMEM((L * chunk,), jnp.float32),
            pltpu.VMEM((256 * L,), jnp.int32),
            pltpu.VMEM((L * (n_cols + 1),), jnp.int32),
            pltpu.VMEM((L,), jnp.float32),
            pltpu.SemaphoreType.DMA,
        ],
    )
    def sel(d2_hbm, out_hbm, buf, hist, cand, tout, sem):
        wid = lax.axis_index("s") * nc + lax.axis_index("c")
        lane = lax.iota(jnp.int32, L)
        zero = jnp.zeros((L,), jnp.int32)
        one = jnp.ones((L,), jnp.int32)
        lane_base = lane * chunk
        seg = n_cols + 1

        def load_chunk(base, ci):
            handles = []
            for l in range(L):
                src = d2_hbm.at[pl.ds((base + l) * n_cols + ci * chunk, chunk)]
                dst = buf.at[pl.ds(l * chunk, chunk)]
                handles.append(pltpu.async_copy(src, dst, sem))
            for h in handles:
                h.wait()

        def group_body(g, _):
            base = wid * rpw + g * L

            @plsc.parallel_loop(0, 256, unroll=8)
            def _(b):
                hist[pl.ds(b * L, L)] = zero

            def hchunk(ci, _):
                load_chunk(base, ci)

                @plsc.parallel_loop(0, chunk, unroll=8)
                def _(cc):
                    col = (cc + lane) & (chunk - 1)
                    v = plsc.load_gather(buf, [lane_base + col])
                    raw = plsc.bitcast(v, jnp.int32)
                    digit = (raw >> 24) & 0xFF
                    plsc.addupdate_scatter(hist, [(digit << 4) + lane], one)
                return 0
            lax.fori_loop(0, nchunk, hchunk, 0)

            def bscan1(t, carry):
                acc, dig, basec = carry
                b = jnp.where(t < 128, 255 - t, t - 128)
                h = hist[pl.ds(b * L, L)]
                acc2 = acc + h
                take = (acc < kk) & (acc2 >= kk)
                dig = jnp.where(take, b, dig)
                basec = jnp.where(take, acc, basec)
                return acc2, dig, basec
            _, dig1, base1 = plsc.parallel_loop(
                0, 256, unroll=4, carry=(zero, zero, zero))(bscan1)
            r = kk - base1
            inv24 = jnp.where(dig1 >= 128, jnp.int32(0x00FFFFFF), zero)

            def cchunk(ci, cnt):
                load_chunk(base, ci)

                def ccol(cc, cnt):
                    col = (cc + lane) & (chunk - 1)
                    v = plsc.load_gather(buf, [lane_base + col])
                    raw = plsc.bitcast(v, jnp.int32)
                    m = ((raw >> 24) & 0xFF) == dig1
                    val = (raw & 0x00FFFFFF) ^ inv24
                    cidx = lane * seg + cnt
                    plsc.store_scatter(cand, [cidx], val, mask=m)
                    return cnt + jnp.where(m, one, zero)
                return plsc.parallel_loop(0, chunk, unroll=4,
                                          carry=cnt)(ccol)
            cnt = lax.fori_loop(0, nchunk, cchunk, zero)

            pref = zero
            for shift in (16, 8, 0):
                @plsc.parallel_loop(0, 256, unroll=8)
                def _(b):
                    hist[pl.ds(b * L, L)] = zero
                maxcnt = jnp.max(cnt)

                def hscan(j):
                    m = j < cnt
                    v = plsc.load_gather(cand, [lane * seg + j], mask=m)
                    digit = (v >> shift) & 0xFF
                    plsc.addupdate_scatter(hist, [(digit << 4) + lane], one,
                                           mask=m)
                plsc.parallel_loop(0, maxcnt, unroll=4)(hscan)

                def bscan2(b, carry):
                    acc, dig, basec = carry
                    h = hist[pl.ds(b * L, L)]
                    acc2 = acc + h
                    take = (acc < r) & (acc2 >= r)
                    dig = jnp.where(take, b, dig)
                    basec = jnp.where(take, acc, basec)
                    return acc2, dig, basec
                _, dig, basec = plsc.parallel_loop(
                    0, 256, unroll=4, carry=(zero, zero, zero))(bscan2)
                r = r - basec
                pref = pref | (dig << shift)

                if shift != 0:
                    def cscan(j, c2):
                        m = j < cnt
                        v = plsc.load_gather(cand, [lane * seg + j], mask=m)
                        keep = m & (((v >> shift) & 0xFF) == dig)
                        plsc.store_scatter(cand, [lane * seg + c2], v,
                                           mask=keep)
                        return c2 + jnp.where(keep, one, zero)
                    cnt = plsc.parallel_loop(0, maxcnt, unroll=4,
                                             carry=zero)(cscan)

            raw_t = (dig1 << 24) | (pref ^ inv24)
            tout[...] = plsc.bitcast(raw_t, jnp.float32)
            pltpu.sync_copy(tout, out_hbm.at[pl.ds(base, L)])
            return 0

        lax.fori_loop(0, groups, group_body, 0)

    return sel(d2.reshape(-1))


def _split_kernel(x_ref, hi_ref, lo_ref):
    x = x_ref[...]
    hi = x.astype(jnp.bfloat16)
    hi_ref[...] = hi
    lo_ref[...] = (x - hi.astype(jnp.float32)).astype(jnp.bfloat16)


def _split(x):
    n, d = x.shape
    return pl.pallas_call(
        _split_kernel,
        grid=(n // _RB,),
        in_specs=[pl.BlockSpec((_RB, d), lambda i: (i, 0))],
        out_specs=[
            pl.BlockSpec((_RB, d), lambda i: (i, 0)),
            pl.BlockSpec((_RB, d), lambda i: (i, 0)),
        ],
        out_shape=[
            jax.ShapeDtypeStruct((n, d), jnp.bfloat16),
            jax.ShapeDtypeStruct((n, d), jnp.bfloat16),
        ],
    )(x)


def _attn_kernel(d2_ref, t_ref, gfhi_ref, gflo_ref, sf_ref, w_ref, out_ref):
    d2 = d2_ref[...]
    mask = jnp.where(d2 <= t_ref[...], 1.0, 0.0)
    maskb = mask.astype(jnp.bfloat16)
    ones = jnp.ones((d2.shape[1], 8), jnp.bfloat16)
    dn = (((1,), (0,)), ((), ()))
    cnt = jax.lax.dot_general(maskb, ones, dn,
                              preferred_element_type=jnp.float32)[:, :1]
    acc = (jax.lax.dot_general(maskb, gfhi_ref[...], dn,
                               preferred_element_type=jnp.float32)
           + jax.lax.dot_general(maskb, gflo_ref[...], dn,
                                 preferred_element_type=jnp.float32))
    mean = acc / cnt
    up = jax.lax.dot_general(mean, w_ref[...], dn,
                             preferred_element_type=jnp.float32, precision=_HI)
    out_ref[...] = sf_ref[...] + up


def _attn_step(d2, t, gfeat_hilo, sfeat, w):
    gfhi, gflo = gfeat_hilo
    n, m = d2.shape
    d = gfhi.shape[1]
    return pl.pallas_call(
        _attn_kernel,
        grid=(n // _RB,),
        in_specs=[
            pl.BlockSpec((_RB, m), lambda i: (i, 0)),
            pl.BlockSpec((_RB, 1), lambda i: (i, 0)),
            pl.BlockSpec((m, d), lambda i: (0, 0)),
            pl.BlockSpec((m, d), lambda i: (0, 0)),
            pl.BlockSpec((_RB, d), lambda i: (i, 0)),
            pl.BlockSpec((d, d), lambda i: (0, 0)),
        ],
        out_specs=pl.BlockSpec((_RB, d), lambda i: (i, 0)),
        out_shape=jax.ShapeDtypeStruct((n, d), jnp.float32),
    )(d2, t, gfhi, gflo, sfeat, w)


def _sim_stats_kernel(tfhi_ref, tflo_ref, afhi_ref, aflo_ref, sim_ref,
                      rmax_ref, rsum_ref, cmax_ref, csum_ref):
    d = tfhi_ref.shape[1]
    dn = (((1,), (1,)), ((), ()))
    tfhi = tfhi_ref[...]
    afhi = afhi_ref[...]
    s = (jax.lax.dot_general(tfhi, afhi, dn,
                             preferred_element_type=jnp.float32)
         + jax.lax.dot_general(tfhi, aflo_ref[...], dn,
                               preferred_element_type=jnp.float32)
         + jax.lax.dot_general(tflo_ref[...], afhi, dn,
                               preferred_element_type=jnp.float32))
    s = s * (1.0 / jnp.sqrt(jnp.float32(d)))
    sim_ref[...] = s
    rmax = jnp.max(s, axis=1, keepdims=True)
    rmax_ref[...] = rmax
    e = jnp.exp(s - rmax)
    ones = jnp.ones((s.shape[1], 8), jnp.float32)
    rsum_ref[...] = jax.lax.dot_general(e, ones, (((1,), (0,)), ((), ())),
                                        preferred_element_type=jnp.float32)[:, :1]
    cmax = jnp.max(s, axis=0)
    cmax_ref[0, 0, :] = cmax
    csum_ref[0, 0, :] = jnp.sum(jnp.exp(s - cmax[None, :]), axis=0)


def _sim_stats(tf_hilo, af_hilo):
    tfhi, tflo = tf_hilo
    afhi, aflo = af_hilo
    n, d = tfhi.shape
    m = afhi.shape[0]
    g = n // _RB
    return pl.pallas_call(
        _sim_stats_kernel,
        grid=(g,),
        in_specs=[
            pl.BlockSpec((_RB, d), lambda i: (i, 0)),
            pl.BlockSpec((_RB, d), lambda i: (i, 0)),
            pl.BlockSpec((m, d), lambda i: (0, 0)),
            pl.BlockSpec((m, d), lambda i: (0, 0)),
        ],
        out_specs=[
            pl.BlockSpec((_RB, m), lambda i: (i, 0)),
            pl.BlockSpec((_RB, 1), lambda i: (i, 0)),
            pl.BlockSpec((_RB, 1), lambda i: (i, 0)),
            pl.BlockSpec((1, 1, m), lambda i: (i, 0, 0)),
            pl.BlockSpec((1, 1, m), lambda i: (i, 0, 0)),
        ],
        out_shape=[
            jax.ShapeDtypeStruct((n, m), jnp.float32),
            jax.ShapeDtypeStruct((n, 1), jnp.float32),
            jax.ShapeDtypeStruct((n, 1), jnp.float32),
            jax.ShapeDtypeStruct((g, 1, m), jnp.float32),
            jax.ShapeDtypeStruct((g, 1, m), jnp.float32),
        ],
    )(tfhi, tflo, afhi, aflo)


def _dual_softmax_kernel(sim_ref, rmax_ref, rsum_ref, cmaxp_ref, csump_ref,
                         out_ref):
    s = sim_ref[...]
    g = cmaxp_ref.shape[0]
    m = cmaxp_ref.shape[2]
    cmaxp = cmaxp_ref[...].reshape(g, m)
    csump = csump_ref[...].reshape(g, m)
    cmax = jnp.max(cmaxp, axis=0)
    csum = jnp.sum(csump * jnp.exp(cmaxp - cmax[None, :]), axis=0)
    num = jnp.exp((s - rmax_ref[...]) + (s - cmax[None, :]))
    out_ref[...] = num / (rsum_ref[...] * csum[None, :])


def _dual_softmax(sim, rmax, rsum, cmaxp, csump):
    n, m = sim.shape
    g = cmaxp.shape[0]
    return pl.pallas_call(
        _dual_softmax_kernel,
        grid=(n // _RB,),
        in_specs=[
            pl.BlockSpec((_RB, m), lambda i: (i, 0)),
            pl.BlockSpec((_RB, 1), lambda i: (i, 0)),
            pl.BlockSpec((_RB, 1), lambda i: (i, 0)),
            pl.BlockSpec((g, 1, m), lambda i: (0, 0, 0)),
            pl.BlockSpec((g, 1, m), lambda i: (0, 0, 0)),
        ],
        out_specs=pl.BlockSpec((_RB, m), lambda i: (i, 0)),
        out_shape=jax.ShapeDtypeStruct((n, m), jnp.float32),
    )(sim, rmax, rsum, cmaxp, csump)


def kernel(anchor_coord, anchor_feat, anchor_offset, target_coord,
           target_feat, target_offset, Wq1, Wc1, Wq2, Wc2):
    tc8 = jnp.pad(target_coord, ((0, 0), (0, 5)))
    ac8 = jnp.pad(anchor_coord, ((0, 0), (0, 5)))

    dtt = _dist(tc8, tc8)
    dta = _dist(tc8, ac8)
    dat = _dist(ac8, tc8)
    t_tt = _sc_select(dtt, _K_QQ).reshape(-1, 1)
    t_ta = _sc_select(dta, _K_QC).reshape(-1, 1)
    t_at = _sc_select(dat, _K_QC).reshape(-1, 1)

    tf = target_feat
    af = anchor_feat
    tf_hl = _split(tf)
    af_hl = _split(af)
    for (wq, wc) in ((Wq1, Wc1), (Wq2, Wc2)):
        tf = _attn_step(dtt, t_tt, tf_hl, tf, wq)
        tf = _attn_step(dta, t_ta, af_hl, tf, wc)
        tf_hl = _split(tf)
        af = _attn_step(dat, t_at, tf_hl, af, wc)
        af_hl = _split(af)

    sim, rmax, rsum, cmaxp, csump = _sim_stats(tf_hl, af_hl)
    return _dual_softmax(sim, rmax, rsum, cmaxp, csump)


# speedup vs baseline: 3.9542x; 1.0616x over previous
import functools

import jax
import jax.numpy as jnp
from jax import lax
from jax.experimental import pallas as pl
from jax.experimental.pallas import tpu as pltpu
from jax.experimental.pallas import tpu_sc as plsc

_K_QQ = 16
_K_QC = 64
_RB = 256
_HI = jax.lax.Precision.HIGHEST


def _monotone_key(x_f32):
    s = jax.lax.bitcast_convert_type(x_f32, jnp.int32)
    return s ^ (jax.lax.shift_right_arithmetic(s, 31) & jnp.int32(0x7FFFFFFF))


def _key_to_float(k_i32):
    s = k_i32 ^ (jax.lax.shift_right_arithmetic(k_i32, 31) & jnp.int32(0x7FFFFFFF))
    return jax.lax.bitcast_convert_type(s, jnp.float32)


def _dist_kernel(q_ref, b_ref, d2_ref):
    q = q_ref[...]
    b = b_ref[...]
    q2 = jnp.sum(q * q, axis=1, keepdims=True)
    b2 = jnp.sum(b * b, axis=1)
    qb = jax.lax.dot_general(q, b, (((1,), (1,)), ((), ())),
                             preferred_element_type=jnp.float32, precision=_HI)
    d2_ref[...] = q2 + b2[None, :] - 2.0 * qb


def _dist(qc8, bc8):
    n = qc8.shape[0]
    m = bc8.shape[0]
    return pl.pallas_call(
        _dist_kernel,
        grid=(n // _RB,),
        in_specs=[
            pl.BlockSpec((_RB, 8), lambda i: (i, 0)),
            pl.BlockSpec((m, 8), lambda i: (0, 0)),
        ],
        out_specs=pl.BlockSpec((_RB, m), lambda i: (i, 0)),
        out_shape=jax.ShapeDtypeStruct((n, m), jnp.float32),
    )(qc8, bc8)


def _sc_select(d2, kk):
    n_rows, n_cols = d2.shape
    info = plsc.get_sparse_core_info()
    nc, ns, L = info.num_cores, info.num_subcores, info.num_lanes
    nw = nc * ns
    rpw = n_rows // nw
    groups = rpw // L
    chunk = 1024
    nchunk = n_cols // chunk
    mesh = plsc.VectorSubcoreMesh(core_axis_name="c", subcore_axis_name="s")

    @functools.partial(
        pl.kernel, mesh=mesh,
        compiler_params=pltpu.CompilerParams(needs_layout_passes=False),
        out_type=jax.ShapeDtypeStruct((n_rows,), jnp.float32),
        scratch_types=[
            pltpu.VMEM((L * chunk,), jnp.float32),
            pltpu.VMEM((L * chunk,), jnp.float32),
            pltpu.VMEM((256 * L,), jnp.int32),
            pltpu.VMEM((L * (n_cols + 1),), jnp.int32),
            pltpu.VMEM((L,), jnp.float32),
            pltpu.SemaphoreType.DMA,
            pltpu.SemaphoreType.DMA,
        ],
    )
    def sel(d2_hbm, out_hbm, bufa, bufb, hist, cand, tout, sema, semb):
        wid = lax.axis_index("s") * nc + lax.axis_index("c")
        lane = lax.iota(jnp.int32, L)
        zero = jnp.zeros((L,), jnp.int32)
        one = jnp.ones((L,), jnp.int32)
        lane_base = lane * chunk
        seg = n_cols + 1

        bufs = (bufa, bufb)
        sems = (sema, semb)

        def start_chunk(base, ci, slot):
            handles = []
            for l in range(L):
                src = d2_hbm.at[pl.ds((base + l) * n_cols + ci * chunk, chunk)]
                dst = bufs[slot].at[pl.ds(l * chunk, chunk)]
                handles.append(pltpu.async_copy(src, dst, sems[slot]))
            return handles

        def scan_chunks(base, scan_one):
            carry = None
            pending = start_chunk(base, 0, 0)
            for ci in range(nchunk):
                nxt = None
                if ci + 1 < nchunk:
                    nxt = start_chunk(base, ci + 1, (ci + 1) % 2)
                for h in pending:
                    h.wait()
                carry = scan_one(bufs[ci % 2], carry)
                pending = nxt
            return carry

        def group_body(g, _):
            base = wid * rpw + g * L

            @plsc.parallel_loop(0, 256, unroll=8)
            def _(b):
                hist[pl.ds(b * L, L)] = zero

            def hscan_chunk(buf, carry):
                @plsc.parallel_loop(0, chunk, unroll=8)
                def _(cc):
                    col = (cc + lane) & (chunk - 1)
                    v = plsc.load_gather(buf, [lane_base + col])
                    raw = plsc.bitcast(v, jnp.int32)
                    digit = (raw >> 24) & 0xFF
                    plsc.addupdate_scatter(hist, [(digit << 4) + lane], one)
                return carry
            scan_chunks(base, hscan_chunk)

            def bscan1(t, carry):
                acc, dig, basec = carry
                b = jnp.where(t < 128, 255 - t, t - 128)
                h = hist[pl.ds(b * L, L)]
                acc2 = acc + h
                take = (acc < kk) & (acc2 >= kk)
                dig = jnp.where(take, b, dig)
                basec = jnp.where(take, acc, basec)
                return acc2, dig, basec
            _, dig1, base1 = plsc.parallel_loop(
                0, 256, unroll=4, carry=(zero, zero, zero))(bscan1)
            r = kk - base1
            inv24 = jnp.where(dig1 >= 128, jnp.int32(0x00FFFFFF), zero)

            def cscan_chunk(buf, cnt):
                if cnt is None:
                    cnt = zero

                def ccol(cc, cnt):
                    col = (cc + lane) & (chunk - 1)
                    v = plsc.load_gather(buf, [lane_base + col])
                    raw = plsc.bitcast(v, jnp.int32)
                    m = ((raw >> 24) & 0xFF) == dig1
                    val = (raw & 0x00FFFFFF) ^ inv24
                    cidx = lane * seg + cnt
                    plsc.store_scatter(cand, [cidx], val, mask=m)
                    return cnt + jnp.where(m, one, zero)
                return plsc.parallel_loop(0, chunk, unroll=4,
                                          carry=cnt)(ccol)
            cnt = scan_chunks(base, cscan_chunk)

            pref = zero
            for shift in (16, 8, 0):
                @plsc.parallel_loop(0, 256, unroll=8)
                def _(b):
                    hist[pl.ds(b * L, L)] = zero
                maxcnt = jnp.max(cnt)

                def hscan(j):
                    m = j < cnt
                    v = plsc.load_gather(cand, [lane * seg + j], mask=m)
                    digit = (v >> shift) & 0xFF
                    plsc.addupdate_scatter(hist, [(digit << 4) + lane], one,
                                           mask=m)
                plsc.parallel_loop(0, maxcnt, unroll=4)(hscan)

                def bscan2(b, carry):
                    acc, dig, basec = carry
                    h = hist[pl.ds(b * L, L)]
                    acc2 = acc + h
                    take = (acc < r) & (acc2 >= r)
                    dig = jnp.where(take, b, dig)
                    basec = jnp.where(take, acc, basec)
                    return acc2, dig, basec
                _, dig, basec = plsc.parallel_loop(
                    0, 256, unroll=4, carry=(zero, zero, zero))(bscan2)
                r = r - basec
                pref = pref | (dig << shift)

                if shift != 0:
                    def cscan(j, c2):
                        m = j < cnt
                        v = plsc.load_gather(cand, [lane * seg + j], mask=m)
                        keep = m & (((v >> shift) & 0xFF) == dig)
                        plsc.store_scatter(cand, [lane * seg + c2], v,
                                           mask=keep)
                        return c2 + jnp.where(keep, one, zero)
                    cnt = plsc.parallel_loop(0, maxcnt, unroll=4,
                                             carry=zero)(cscan)

            raw_t = (dig1 << 24) | (pref ^ inv24)
            tout[...] = plsc.bitcast(raw_t, jnp.float32)
            pltpu.sync_copy(tout, out_hbm.at[pl.ds(base, L)])
            return 0

        lax.fori_loop(0, groups, group_body, 0)

    return sel(d2.reshape(-1))


def _split_kernel(x_ref, hi_ref, lo_ref):
    x = x_ref[...]
    hi = x.astype(jnp.bfloat16)
    hi_ref[...] = hi
    lo_ref[...] = (x - hi.astype(jnp.float32)).astype(jnp.bfloat16)


def _split(x):
    n, d = x.shape
    return pl.pallas_call(
        _split_kernel,
        grid=(n // _RB,),
        in_specs=[pl.BlockSpec((_RB, d), lambda i: (i, 0))],
        out_specs=[
            pl.BlockSpec((_RB, d), lambda i: (i, 0)),
            pl.BlockSpec((_RB, d), lambda i: (i, 0)),
        ],
        out_shape=[
            jax.ShapeDtypeStruct((n, d), jnp.bfloat16),
            jax.ShapeDtypeStruct((n, d), jnp.bfloat16),
        ],
    )(x)


def _attn_kernel(d2_ref, t_ref, gfhi_ref, gflo_ref, sf_ref, w_ref, out_ref):
    d2 = d2_ref[...]
    mask = jnp.where(d2 <= t_ref[...], 1.0, 0.0)
    maskb = mask.astype(jnp.bfloat16)
    ones = jnp.ones((d2.shape[1], 8), jnp.bfloat16)
    dn = (((1,), (0,)), ((), ()))
    cnt = jax.lax.dot_general(maskb, ones, dn,
                              preferred_element_type=jnp.float32)[:, :1]
    acc = (jax.lax.dot_general(maskb, gfhi_ref[...], dn,
                               preferred_element_type=jnp.float32)
           + jax.lax.dot_general(maskb, gflo_ref[...], dn,
                                 preferred_element_type=jnp.float32))
    mean = acc / cnt
    up = jax.lax.dot_general(mean, w_ref[...], dn,
                             preferred_element_type=jnp.float32, precision=_HI)
    out_ref[...] = sf_ref[...] + up


def _attn_step(d2, t, gfeat_hilo, sfeat, w):
    gfhi, gflo = gfeat_hilo
    n, m = d2.shape
    d = gfhi.shape[1]
    return pl.pallas_call(
        _attn_kernel,
        grid=(n // _RB,),
        in_specs=[
            pl.BlockSpec((_RB, m), lambda i: (i, 0)),
            pl.BlockSpec((_RB, 1), lambda i: (i, 0)),
            pl.BlockSpec((m, d), lambda i: (0, 0)),
            pl.BlockSpec((m, d), lambda i: (0, 0)),
            pl.BlockSpec((_RB, d), lambda i: (i, 0)),
            pl.BlockSpec((d, d), lambda i: (0, 0)),
        ],
        out_specs=pl.BlockSpec((_RB, d), lambda i: (i, 0)),
        out_shape=jax.ShapeDtypeStruct((n, d), jnp.float32),
    )(d2, t, gfhi, gflo, sfeat, w)


def _sim_stats_kernel(tfhi_ref, tflo_ref, afhi_ref, aflo_ref, sim_ref,
                      rmax_ref, rsum_ref, cmax_ref, csum_ref):
    d = tfhi_ref.shape[1]
    dn = (((1,), (1,)), ((), ()))
    tfhi = tfhi_ref[...]
    afhi = afhi_ref[...]
    s = (jax.lax.dot_general(tfhi, afhi, dn,
                             preferred_element_type=jnp.float32)
         + jax.lax.dot_general(tfhi, aflo_ref[...], dn,
                               preferred_element_type=jnp.float32)
         + jax.lax.dot_general(tflo_ref[...], afhi, dn,
                               preferred_element_type=jnp.float32))
    s = s * (1.0 / jnp.sqrt(jnp.float32(d)))
    sim_ref[...] = s
    rmax = jnp.max(s, axis=1, keepdims=True)
    rmax_ref[...] = rmax
    e = jnp.exp(s - rmax)
    ones = jnp.ones((s.shape[1], 8), jnp.float32)
    rsum_ref[...] = jax.lax.dot_general(e, ones, (((1,), (0,)), ((), ())),
                                        preferred_element_type=jnp.float32)[:, :1]
    cmax = jnp.max(s, axis=0)
    cmax_ref[0, 0, :] = cmax
    csum_ref[0, 0, :] = jnp.sum(jnp.exp(s - cmax[None, :]), axis=0)


def _sim_stats(tf_hilo, af_hilo):
    tfhi, tflo = tf_hilo
    afhi, aflo = af_hilo
    n, d = tfhi.shape
    m = afhi.shape[0]
    g = n // _RB
    return pl.pallas_call(
        _sim_stats_kernel,
        grid=(g,),
        in_specs=[
            pl.BlockSpec((_RB, d), lambda i: (i, 0)),
            pl.BlockSpec((_RB, d), lambda i: (i, 0)),
            pl.BlockSpec((m, d), lambda i: (0, 0)),
            pl.BlockSpec((m, d), lambda i: (0, 0)),
        ],
        out_specs=[
            pl.BlockSpec((_RB, m), lambda i: (i, 0)),
            pl.BlockSpec((_RB, 1), lambda i: (i, 0)),
            pl.BlockSpec((_RB, 1), lambda i: (i, 0)),
            pl.BlockSpec((1, 1, m), lambda i: (i, 0, 0)),
            pl.BlockSpec((1, 1, m), lambda i: (i, 0, 0)),
        ],
        out_shape=[
            jax.ShapeDtypeStruct((n, m), jnp.float32),
            jax.ShapeDtypeStruct((n, 1), jnp.float32),
            jax.ShapeDtypeStruct((n, 1), jnp.float32),
            jax.ShapeDtypeStruct((g, 1, m), jnp.float32),
            jax.ShapeDtypeStruct((g, 1, m), jnp.float32),
        ],
    )(tfhi, tflo, afhi, aflo)


def _dual_softmax_kernel(sim_ref, rmax_ref, rsum_ref, cmaxp_ref, csump_ref,
                         out_ref):
    s = sim_ref[...]
    g = cmaxp_ref.shape[0]
    m = cmaxp_ref.shape[2]
    cmaxp = cmaxp_ref[...].reshape(g, m)
    csump = csump_ref[...].reshape(g, m)
    cmax = jnp.max(cmaxp, axis=0)
    csum = jnp.sum(csump * jnp.exp(cmaxp - cmax[None, :]), axis=0)
    num = jnp.exp((s - rmax_ref[...]) + (s - cmax[None, :]))
    out_ref[...] = num / (rsum_ref[...] * csum[None, :])


def _dual_softmax(sim, rmax, rsum, cmaxp, csump):
    n, m = sim.shape
    g = cmaxp.shape[0]
    return pl.pallas_call(
        _dual_softmax_kernel,
        grid=(n // _RB,),
        in_specs=[
            pl.BlockSpec((_RB, m), lambda i: (i, 0)),
            pl.BlockSpec((_RB, 1), lambda i: (i, 0)),
            pl.BlockSpec((_RB, 1), lambda i: (i, 0)),
            pl.BlockSpec((g, 1, m), lambda i: (0, 0, 0)),
            pl.BlockSpec((g, 1, m), lambda i: (0, 0, 0)),
        ],
        out_specs=pl.BlockSpec((_RB, m), lambda i: (i, 0)),
        out_shape=jax.ShapeDtypeStruct((n, m), jnp.float32),
    )(sim, rmax, rsum, cmaxp, csump)


def kernel(anchor_coord, anchor_feat, anchor_offset, target_coord,
           target_feat, target_offset, Wq1, Wc1, Wq2, Wc2):
    tc8 = jnp.pad(target_coord, ((0, 0), (0, 5)))
    ac8 = jnp.pad(anchor_coord, ((0, 0), (0, 5)))

    dtt = _dist(tc8, tc8)
    dta = _dist(tc8, ac8)
    dat = _dist(ac8, tc8)
    t_tt = _sc_select(dtt, _K_QQ).reshape(-1, 1)
    t_ta = _sc_select(dta, _K_QC).reshape(-1, 1)
    t_at = _sc_select(dat, _K_QC).reshape(-1, 1)

    tf = target_feat
    af = anchor_feat
    tf_hl = _split(tf)
    af_hl = _split(af)
    for (wq, wc) in ((Wq1, Wc1), (Wq2, Wc2)):
        tf = _attn_step(dtt, t_tt, tf_hl, tf, wq)
        tf = _attn_step(dta, t_ta, af_hl, tf, wc)
        tf_hl = _split(tf)
        af = _attn_step(dat, t_at, tf_hl, af, wc)
        af_hl = _split(af)

    sim, rmax, rsum, cmaxp, csump = _sim_stats(tf_hl, af_hl)
    return _dual_softmax(sim, rmax, rsum, cmaxp, csump)


# speedup vs baseline: 4.0138x; 1.0151x over previous
import functools

import jax
import jax.numpy as jnp
from jax import lax
from jax.experimental import pallas as pl
from jax.experimental.pallas import tpu as pltpu
from jax.experimental.pallas import tpu_sc as plsc

_K_QQ = 16
_K_QC = 64
_RB = 256
_HI = jax.lax.Precision.HIGHEST


def _monotone_key(x_f32):
    s = jax.lax.bitcast_convert_type(x_f32, jnp.int32)
    return s ^ (jax.lax.shift_right_arithmetic(s, 31) & jnp.int32(0x7FFFFFFF))


def _key_to_float(k_i32):
    s = k_i32 ^ (jax.lax.shift_right_arithmetic(k_i32, 31) & jnp.int32(0x7FFFFFFF))
    return jax.lax.bitcast_convert_type(s, jnp.float32)


def _dist_kernel(q_ref, b_ref, d2_ref):
    q = q_ref[...]
    b = b_ref[...]
    q2 = jnp.sum(q * q, axis=1, keepdims=True)
    b2 = jnp.sum(b * b, axis=1)
    qb = jax.lax.dot_general(q, b, (((1,), (1,)), ((), ())),
                             preferred_element_type=jnp.float32, precision=_HI)
    d2_ref[...] = q2 + b2[None, :] - 2.0 * qb


def _dist(qc8, bc8):
    n = qc8.shape[0]
    m = bc8.shape[0]
    return pl.pallas_call(
        _dist_kernel,
        grid=(n // _RB,),
        in_specs=[
            pl.BlockSpec((_RB, 8), lambda i: (i, 0)),
            pl.BlockSpec((m, 8), lambda i: (0, 0)),
        ],
        out_specs=pl.BlockSpec((_RB, m), lambda i: (i, 0)),
        out_shape=jax.ShapeDtypeStruct((n, m), jnp.float32),
    )(qc8, bc8)


def _sc_select(d2, kk):
    n_rows, n_cols = d2.shape
    info = plsc.get_sparse_core_info()
    nc, ns, L = info.num_cores, info.num_subcores, info.num_lanes
    nw = nc * ns
    rpw = n_rows // nw
    groups = rpw // L
    chunk = 1024
    nchunk = n_cols // chunk
    mesh = plsc.VectorSubcoreMesh(core_axis_name="c", subcore_axis_name="s")

    @functools.partial(
        pl.kernel, mesh=mesh,
        compiler_params=pltpu.CompilerParams(needs_layout_passes=False),
        out_type=jax.ShapeDtypeStruct((n_rows,), jnp.float32),
        scratch_types=[
            pltpu.VMEM((L * chunk,), jnp.float32),
            pltpu.VMEM((L * chunk,), jnp.float32),
            pltpu.VMEM((256 * L,), jnp.int32),
            pltpu.VMEM((L * (n_cols + 1),), jnp.int32),
            pltpu.VMEM((L,), jnp.float32),
            pltpu.SemaphoreType.DMA,
            pltpu.SemaphoreType.DMA,
        ],
    )
    def sel(d2_hbm, out_hbm, bufa, bufb, hist, cand, tout, sema, semb):
        wid = lax.axis_index("s") * nc + lax.axis_index("c")
        lane = lax.iota(jnp.int32, L)
        zero = jnp.zeros((L,), jnp.int32)
        one = jnp.ones((L,), jnp.int32)
        lane_base = lane * chunk
        seg = n_cols + 1

        bufs = (bufa, bufb)
        sems = (sema, semb)

        def start_chunk(base, ci, slot):
            handles = []
            for l in range(L):
                src = d2_hbm.at[pl.ds((base + l) * n_cols + ci * chunk, chunk)]
                dst = bufs[slot].at[pl.ds(l * chunk, chunk)]
                handles.append(pltpu.async_copy(src, dst, sems[slot]))
            return handles

        def scan_chunks(base, scan_one):
            carry = None
            pending = start_chunk(base, 0, 0)
            for ci in range(nchunk):
                nxt = None
                if ci + 1 < nchunk:
                    nxt = start_chunk(base, ci + 1, (ci + 1) % 2)
                for h in pending:
                    h.wait()
                carry = scan_one(bufs[ci % 2], carry)
                pending = nxt
            return carry

        def group_body(g, _):
            base = wid * rpw + g * L

            @plsc.parallel_loop(0, 256, unroll=8)
            def _(b):
                hist[pl.ds(b * L, L)] = zero

            def hscan_chunk(buf, carry):
                @plsc.parallel_loop(0, chunk, unroll=8)
                def _(cc):
                    col = (cc + lane) & (chunk - 1)
                    v = plsc.load_gather(buf, [lane_base + col])
                    raw = plsc.bitcast(v, jnp.int32)
                    digit = (raw >> 24) & 0xFF
                    plsc.addupdate_scatter(hist, [(digit << 4) + lane], one)
                return carry
            scan_chunks(base, hscan_chunk)

            def bscan1(t, carry):
                acc, dig, basec = carry
                b = jnp.where(t < 128, 255 - t, t - 128)
                h = hist[pl.ds(b * L, L)]
                acc2 = acc + h
                take = (acc < kk) & (acc2 >= kk)
                dig = jnp.where(take, b, dig)
                basec = jnp.where(take, acc, basec)
                return acc2, dig, basec
            _, dig1, base1 = plsc.parallel_loop(
                0, 256, unroll=4, carry=(zero, zero, zero))(bscan1)
            r = kk - base1
            inv24 = jnp.where(dig1 >= 128, jnp.int32(0x00FFFFFF), zero)

            def cscan_chunk(buf, cnt):
                if cnt is None:
                    cnt = zero

                def ccol(cc, cnt):
                    col = (cc + lane) & (chunk - 1)
                    v = plsc.load_gather(buf, [lane_base + col])
                    raw = plsc.bitcast(v, jnp.int32)
                    m = ((raw >> 24) & 0xFF) == dig1
                    val = (raw & 0x00FFFFFF) ^ inv24
                    cidx = lane * seg + cnt
                    plsc.store_scatter(cand, [cidx], val, mask=m)
                    return cnt + jnp.where(m, one, zero)
                return plsc.parallel_loop(0, chunk, unroll=4,
                                          carry=cnt)(ccol)
            cnt = scan_chunks(base, cscan_chunk)

            pref = zero
            for shift in (16, 8, 0):
                @plsc.parallel_loop(0, 256, unroll=8)
                def _(b):
                    hist[pl.ds(b * L, L)] = zero
                maxcnt = jnp.max(cnt)

                def hscan(j):
                    m = j < cnt
                    v = plsc.load_gather(cand, [lane * seg + j], mask=m)
                    digit = (v >> shift) & 0xFF
                    plsc.addupdate_scatter(hist, [(digit << 4) + lane], one,
                                           mask=m)
                plsc.parallel_loop(0, maxcnt, unroll=4)(hscan)

                def bscan2(b, carry):
                    acc, dig, basec = carry
                    h = hist[pl.ds(b * L, L)]
                    acc2 = acc + h
                    take = (acc < r) & (acc2 >= r)
                    dig = jnp.where(take, b, dig)
                    basec = jnp.where(take, acc, basec)
                    return acc2, dig, basec
                _, dig, basec = plsc.parallel_loop(
                    0, 256, unroll=4, carry=(zero, zero, zero))(bscan2)
                r = r - basec
                pref = pref | (dig << shift)

                if shift != 0:
                    def cscan(j, c2):
                        m = j < cnt
                        v = plsc.load_gather(cand, [lane * seg + j], mask=m)
                        keep = m & (((v >> shift) & 0xFF) == dig)
                        plsc.store_scatter(cand, [lane * seg + c2], v,
                                           mask=keep)
                        return c2 + jnp.where(keep, one, zero)
                    cnt = plsc.parallel_loop(0, maxcnt, unroll=4,
                                             carry=zero)(cscan)

            raw_t = (dig1 << 24) | (pref ^ inv24)
            tout[...] = plsc.bitcast(raw_t, jnp.float32)
            pltpu.sync_copy(tout, out_hbm.at[pl.ds(base, L)])
            return 0

        lax.fori_loop(0, groups, group_body, 0)

    return sel(d2.reshape(-1))


def _split_kernel(x_ref, hi_ref, lo_ref):
    x = x_ref[...]
    hi = x.astype(jnp.bfloat16)
    hi_ref[...] = hi
    lo_ref[...] = (x - hi.astype(jnp.float32)).astype(jnp.bfloat16)


def _split(x):
    n, d = x.shape
    return pl.pallas_call(
        _split_kernel,
        grid=(n // _RB,),
        in_specs=[pl.BlockSpec((_RB, d), lambda i: (i, 0))],
        out_specs=[
            pl.BlockSpec((_RB, d), lambda i: (i, 0)),
            pl.BlockSpec((_RB, d), lambda i: (i, 0)),
        ],
        out_shape=[
            jax.ShapeDtypeStruct((n, d), jnp.bfloat16),
            jax.ShapeDtypeStruct((n, d), jnp.bfloat16),
        ],
    )(x)


def _attn_kernel(d2_ref, t_ref, gfhi_ref, gflo_ref, sf_ref, w_ref, out_ref):
    d2 = d2_ref[...]
    mask = jnp.where(d2 <= t_ref[...], 1.0, 0.0)
    maskb = mask.astype(jnp.bfloat16)
    dn = (((1,), (0,)), ((), ()))
    cnt = jnp.sum(mask, axis=1, keepdims=True)
    acc = (jax.lax.dot_general(maskb, gfhi_ref[...], dn,
                               preferred_element_type=jnp.float32)
           + jax.lax.dot_general(maskb, gflo_ref[...], dn,
                                 preferred_element_type=jnp.float32))
    mean = acc / cnt
    up = jax.lax.dot_general(mean, w_ref[...], dn,
                             preferred_element_type=jnp.float32, precision=_HI)
    out_ref[...] = sf_ref[...] + up


def _attn_step(d2, t, gfeat_hilo, sfeat, w):
    gfhi, gflo = gfeat_hilo
    n, m = d2.shape
    d = gfhi.shape[1]
    return pl.pallas_call(
        _attn_kernel,
        grid=(n // _RB,),
        in_specs=[
            pl.BlockSpec((_RB, m), lambda i: (i, 0)),
            pl.BlockSpec((_RB, 1), lambda i: (i, 0)),
            pl.BlockSpec((m, d), lambda i: (0, 0)),
            pl.BlockSpec((m, d), lambda i: (0, 0)),
            pl.BlockSpec((_RB, d), lambda i: (i, 0)),
            pl.BlockSpec((d, d), lambda i: (0, 0)),
        ],
        out_specs=pl.BlockSpec((_RB, d), lambda i: (i, 0)),
        out_shape=jax.ShapeDtypeStruct((n, d), jnp.float32),
    )(d2, t, gfhi, gflo, sfeat, w)


def _sim_stats_kernel(tfhi_ref, tflo_ref, afhi_ref, aflo_ref, sim_ref,
                      rmax_ref, rsum_ref, cmax_ref, csum_ref):
    d = tfhi_ref.shape[1]
    dn = (((1,), (1,)), ((), ()))
    tfhi = tfhi_ref[...]
    afhi = afhi_ref[...]
    s = (jax.lax.dot_general(tfhi, afhi, dn,
                             preferred_element_type=jnp.float32)
         + jax.lax.dot_general(tfhi, aflo_ref[...], dn,
                               preferred_element_type=jnp.float32)
         + jax.lax.dot_general(tflo_ref[...], afhi, dn,
                               preferred_element_type=jnp.float32))
    s = s * (1.0 / jnp.sqrt(jnp.float32(d)))
    sim_ref[...] = s
    rmax = jnp.max(s, axis=1, keepdims=True)
    rmax_ref[...] = rmax
    e = jnp.exp(s - rmax)
    rsum_ref[...] = jnp.sum(e, axis=1, keepdims=True)
    cmax = jnp.max(s, axis=0)
    cmax_ref[0, 0, :] = cmax
    csum_ref[0, 0, :] = jnp.sum(jnp.exp(s - cmax[None, :]), axis=0)


def _sim_stats(tf_hilo, af_hilo):
    tfhi, tflo = tf_hilo
    afhi, aflo = af_hilo
    n, d = tfhi.shape
    m = afhi.shape[0]
    g = n // _RB
    return pl.pallas_call(
        _sim_stats_kernel,
        grid=(g,),
        in_specs=[
            pl.BlockSpec((_RB, d), lambda i: (i, 0)),
            pl.BlockSpec((_RB, d), lambda i: (i, 0)),
            pl.BlockSpec((m, d), lambda i: (0, 0)),
            pl.BlockSpec((m, d), lambda i: (0, 0)),
        ],
        out_specs=[
            pl.BlockSpec((_RB, m), lambda i: (i, 0)),
            pl.BlockSpec((_RB, 1), lambda i: (i, 0)),
            pl.BlockSpec((_RB, 1), lambda i: (i, 0)),
            pl.BlockSpec((1, 1, m), lambda i: (i, 0, 0)),
            pl.BlockSpec((1, 1, m), lambda i: (i, 0, 0)),
        ],
        out_shape=[
            jax.ShapeDtypeStruct((n, m), jnp.float32),
            jax.ShapeDtypeStruct((n, 1), jnp.float32),
            jax.ShapeDtypeStruct((n, 1), jnp.float32),
            jax.ShapeDtypeStruct((g, 1, m), jnp.float32),
            jax.ShapeDtypeStruct((g, 1, m), jnp.float32),
        ],
    )(tfhi, tflo, afhi, aflo)


def _dual_softmax_kernel(sim_ref, rmax_ref, rsum_ref, cmaxp_ref, csump_ref,
                         out_ref):
    s = sim_ref[...]
    g = cmaxp_ref.shape[0]
    m = cmaxp_ref.shape[2]
    cmaxp = cmaxp_ref[...].reshape(g, m)
    csump = csump_ref[...].reshape(g, m)
    cmax = jnp.max(cmaxp, axis=0)
    csum = jnp.sum(csump * jnp.exp(cmaxp - cmax[None, :]), axis=0)
    num = jnp.exp((s - rmax_ref[...]) + (s - cmax[None, :]))
    out_ref[...] = num / (rsum_ref[...] * csum[None, :])


def _dual_softmax(sim, rmax, rsum, cmaxp, csump):
    n, m = sim.shape
    g = cmaxp.shape[0]
    return pl.pallas_call(
        _dual_softmax_kernel,
        grid=(n // _RB,),
        in_specs=[
            pl.BlockSpec((_RB, m), lambda i: (i, 0)),
            pl.BlockSpec((_RB, 1), lambda i: (i, 0)),
            pl.BlockSpec((_RB, 1), lambda i: (i, 0)),
            pl.BlockSpec((g, 1, m), lambda i: (0, 0, 0)),
            pl.BlockSpec((g, 1, m), lambda i: (0, 0, 0)),
        ],
        out_specs=pl.BlockSpec((_RB, m), lambda i: (i, 0)),
        out_shape=jax.ShapeDtypeStruct((n, m), jnp.float32),
    )(sim, rmax, rsum, cmaxp, csump)


def kernel(anchor_coord, anchor_feat, anchor_offset, target_coord,
           target_feat, target_offset, Wq1, Wc1, Wq2, Wc2):
    tc8 = jnp.pad(target_coord, ((0, 0), (0, 5)))
    ac8 = jnp.pad(anchor_coord, ((0, 0), (0, 5)))

    dtt = _dist(tc8, tc8)
    dta = _dist(tc8, ac8)
    dat = _dist(ac8, tc8)
    t_tt = _sc_select(dtt, _K_QQ).reshape(-1, 1)
    t_ta = _sc_select(dta, _K_QC).reshape(-1, 1)
    t_at = _sc_select(dat, _K_QC).reshape(-1, 1)

    tf = target_feat
    af = anchor_feat
    tf_hl = _split(tf)
    af_hl = _split(af)
    for (wq, wc) in ((Wq1, Wc1), (Wq2, Wc2)):
        tf = _attn_step(dtt, t_tt, tf_hl, tf, wq)
        tf = _attn_step(dta, t_ta, af_hl, tf, wc)
        tf_hl = _split(tf)
        af = _attn_step(dat, t_at, tf_hl, af, wc)
        af_hl = _split(af)

    sim, rmax, rsum, cmaxp, csump = _sim_stats(tf_hl, af_hl)
    return _dual_softmax(sim, rmax, rsum, cmaxp, csump)


# speedup vs baseline: 4.8868x; 1.2175x over previous
import functools

import jax
import jax.numpy as jnp
from jax import lax
from jax.experimental import pallas as pl
from jax.experimental.pallas import tpu as pltpu
from jax.experimental.pallas import tpu_sc as plsc

_K_QQ = 16
_K_QC = 64
_RB = 256
_HI = jax.lax.Precision.HIGHEST


def _monotone_key(x_f32):
    s = jax.lax.bitcast_convert_type(x_f32, jnp.int32)
    return s ^ (jax.lax.shift_right_arithmetic(s, 31) & jnp.int32(0x7FFFFFFF))


def _key_to_float(k_i32):
    s = k_i32 ^ (jax.lax.shift_right_arithmetic(k_i32, 31) & jnp.int32(0x7FFFFFFF))
    return jax.lax.bitcast_convert_type(s, jnp.float32)


def _dist_kernel(q_ref, b_ref, d2_ref):
    q = q_ref[...]
    b = b_ref[...]
    q2 = jnp.sum(q * q, axis=1, keepdims=True)
    b2 = jnp.sum(b * b, axis=1)
    qb = jax.lax.dot_general(q, b, (((1,), (1,)), ((), ())),
                             preferred_element_type=jnp.float32, precision=_HI)
    d2_ref[...] = q2 + b2[None, :] - 2.0 * qb


def _dist(qc8, bc8):
    n = qc8.shape[0]
    m = bc8.shape[0]
    return pl.pallas_call(
        _dist_kernel,
        grid=(n // _RB,),
        in_specs=[
            pl.BlockSpec((_RB, 8), lambda i: (i, 0)),
            pl.BlockSpec((m, 8), lambda i: (0, 0)),
        ],
        out_specs=pl.BlockSpec((_RB, m), lambda i: (i, 0)),
        out_shape=jax.ShapeDtypeStruct((n, m), jnp.float32),
    )(qc8, bc8)


def _sc_select(d2, kk):
    n_rows, n_cols = d2.shape
    info = plsc.get_sparse_core_info()
    nc, ns, L = info.num_cores, info.num_subcores, info.num_lanes
    nw = nc * ns
    rpw = n_rows // nw
    groups = rpw // L
    chunk = 1024
    nchunk = n_cols // chunk
    mesh = plsc.VectorSubcoreMesh(core_axis_name="c", subcore_axis_name="s")

    @functools.partial(
        pl.kernel, mesh=mesh,
        compiler_params=pltpu.CompilerParams(needs_layout_passes=False),
        out_type=jax.ShapeDtypeStruct((n_rows,), jnp.float32),
        scratch_types=[
            pltpu.VMEM((L * chunk,), jnp.float32),
            pltpu.VMEM((L * chunk,), jnp.float32),
            pltpu.VMEM((256 * L,), jnp.int32),
            pltpu.VMEM((L * (n_cols + 1),), jnp.int32),
            pltpu.VMEM((L,), jnp.float32),
            pltpu.SemaphoreType.DMA,
            pltpu.SemaphoreType.DMA,
        ],
    )
    def sel(d2_hbm, out_hbm, bufa, bufb, hist, cand, tout, sema, semb):
        wid = lax.axis_index("s") * nc + lax.axis_index("c")
        lane = lax.iota(jnp.int32, L)
        zero = jnp.zeros((L,), jnp.int32)
        one = jnp.ones((L,), jnp.int32)
        lane_base = lane * chunk
        seg = n_cols + 1

        bufs = (bufa, bufb)
        sems = (sema, semb)

        def start_chunk(base, ci, slot):
            handles = []
            for l in range(L):
                src = d2_hbm.at[base + l, pl.ds(ci * chunk, chunk)]
                dst = bufs[slot].at[pl.ds(l * chunk, chunk)]
                handles.append(pltpu.async_copy(src, dst, sems[slot]))
            return handles

        def scan_chunks(base, scan_one):
            carry = None
            pending = start_chunk(base, 0, 0)
            for ci in range(nchunk):
                nxt = None
                if ci + 1 < nchunk:
                    nxt = start_chunk(base, ci + 1, (ci + 1) % 2)
                for h in pending:
                    h.wait()
                carry = scan_one(bufs[ci % 2], carry)
                pending = nxt
            return carry

        def group_body(g, _):
            base = wid * rpw + g * L

            @plsc.parallel_loop(0, 256, unroll=8)
            def _(b):
                hist[pl.ds(b * L, L)] = zero

            def hscan_chunk(buf, carry):
                @plsc.parallel_loop(0, chunk, unroll=8)
                def _(cc):
                    col = (cc + lane) & (chunk - 1)
                    v = plsc.load_gather(buf, [lane_base + col])
                    raw = plsc.bitcast(v, jnp.int32)
                    digit = (raw >> 24) & 0xFF
                    plsc.addupdate_scatter(hist, [(digit << 4) + lane], one)
                return carry
            scan_chunks(base, hscan_chunk)

            def bscan1(t, carry):
                acc, dig, basec = carry
                b = jnp.where(t < 128, 255 - t, t - 128)
                h = hist[pl.ds(b * L, L)]
                acc2 = acc + h
                take = (acc < kk) & (acc2 >= kk)
                dig = jnp.where(take, b, dig)
                basec = jnp.where(take, acc, basec)
                return acc2, dig, basec
            _, dig1, base1 = plsc.parallel_loop(
                0, 256, unroll=4, carry=(zero, zero, zero))(bscan1)
            r = kk - base1
            inv24 = jnp.where(dig1 >= 128, jnp.int32(0x00FFFFFF), zero)

            def cscan_chunk(buf, cnt):
                if cnt is None:
                    cnt = zero

                def ccol(cc, cnt):
                    col = (cc + lane) & (chunk - 1)
                    v = plsc.load_gather(buf, [lane_base + col])
                    raw = plsc.bitcast(v, jnp.int32)
                    m = ((raw >> 24) & 0xFF) == dig1
                    val = (raw & 0x00FFFFFF) ^ inv24
                    cidx = lane * seg + cnt
                    plsc.store_scatter(cand, [cidx], val, mask=m)
                    return cnt + jnp.where(m, one, zero)
                return plsc.parallel_loop(0, chunk, unroll=4,
                                          carry=cnt)(ccol)
            cnt = scan_chunks(base, cscan_chunk)

            pref = zero
            for shift in (16, 8, 0):
                @plsc.parallel_loop(0, 256, unroll=8)
                def _(b):
                    hist[pl.ds(b * L, L)] = zero
                maxcnt = jnp.max(cnt)

                def hscan(j):
                    m = j < cnt
                    v = plsc.load_gather(cand, [lane * seg + j], mask=m)
                    digit = (v >> shift) & 0xFF
                    plsc.addupdate_scatter(hist, [(digit << 4) + lane], one,
                                           mask=m)
                plsc.parallel_loop(0, maxcnt, unroll=4)(hscan)

                def bscan2(b, carry):
                    acc, dig, basec = carry
                    h = hist[pl.ds(b * L, L)]
                    acc2 = acc + h
                    take = (acc < r) & (acc2 >= r)
                    dig = jnp.where(take, b, dig)
                    basec = jnp.where(take, acc, basec)
                    return acc2, dig, basec
                _, dig, basec = plsc.parallel_loop(
                    0, 256, unroll=4, carry=(zero, zero, zero))(bscan2)
                r = r - basec
                pref = pref | (dig << shift)

                if shift != 0:
                    def cscan(j, c2):
                        m = j < cnt
                        v = plsc.load_gather(cand, [lane * seg + j], mask=m)
                        keep = m & (((v >> shift) & 0xFF) == dig)
                        plsc.store_scatter(cand, [lane * seg + c2], v,
                                           mask=keep)
                        return c2 + jnp.where(keep, one, zero)
                    cnt = plsc.parallel_loop(0, maxcnt, unroll=4,
                                             carry=zero)(cscan)

            raw_t = (dig1 << 24) | (pref ^ inv24)
            tout[...] = plsc.bitcast(raw_t, jnp.float32)
            pltpu.sync_copy(tout, out_hbm.at[pl.ds(base, L)])
            return 0

        lax.fori_loop(0, groups, group_body, 0)

    return sel(d2)


def _split_kernel(x_ref, hi_ref, lo_ref):
    x = x_ref[...]
    hi = x.astype(jnp.bfloat16)
    hi_ref[...] = hi
    lo_ref[...] = (x - hi.astype(jnp.float32)).astype(jnp.bfloat16)


def _split(x):
    n, d = x.shape
    return pl.pallas_call(
        _split_kernel,
        grid=(n // _RB,),
        in_specs=[pl.BlockSpec((_RB, d), lambda i: (i, 0))],
        out_specs=[
            pl.BlockSpec((_RB, d), lambda i: (i, 0)),
            pl.BlockSpec((_RB, d), lambda i: (i, 0)),
        ],
        out_shape=[
            jax.ShapeDtypeStruct((n, d), jnp.bfloat16),
            jax.ShapeDtypeStruct((n, d), jnp.bfloat16),
        ],
    )(x)


def _attn_kernel(d2_ref, t_ref, gfhi_ref, gflo_ref, sf_ref, w_ref, out_ref):
    d2 = d2_ref[...]
    mask = jnp.where(d2 <= t_ref[...], 1.0, 0.0)
    maskb = mask.astype(jnp.bfloat16)
    dn = (((1,), (0,)), ((), ()))
    cnt = jnp.sum(mask, axis=1, keepdims=True)
    acc = (jax.lax.dot_general(maskb, gfhi_ref[...], dn,
                               preferred_element_type=jnp.float32)
           + jax.lax.dot_general(maskb, gflo_ref[...], dn,
                                 preferred_element_type=jnp.float32))
    mean = acc / cnt
    up = jax.lax.dot_general(mean, w_ref[...], dn,
                             preferred_element_type=jnp.float32, precision=_HI)
    out_ref[...] = sf_ref[...] + up


def _attn_step(d2, t, gfeat_hilo, sfeat, w):
    gfhi, gflo = gfeat_hilo
    n, m = d2.shape
    d = gfhi.shape[1]
    return pl.pallas_call(
        _attn_kernel,
        grid=(n // _RB,),
        in_specs=[
            pl.BlockSpec((_RB, m), lambda i: (i, 0)),
            pl.BlockSpec((_RB, 1), lambda i: (i, 0)),
            pl.BlockSpec((m, d), lambda i: (0, 0)),
            pl.BlockSpec((m, d), lambda i: (0, 0)),
            pl.BlockSpec((_RB, d), lambda i: (i, 0)),
            pl.BlockSpec((d, d), lambda i: (0, 0)),
        ],
        out_specs=pl.BlockSpec((_RB, d), lambda i: (i, 0)),
        out_shape=jax.ShapeDtypeStruct((n, d), jnp.float32),
    )(d2, t, gfhi, gflo, sfeat, w)


def _sim_stats_kernel(tfhi_ref, tflo_ref, afhi_ref, aflo_ref, sim_ref,
                      rmax_ref, rsum_ref, cmax_ref, csum_ref):
    d = tfhi_ref.shape[1]
    dn = (((1,), (1,)), ((), ()))
    tfhi = tfhi_ref[...]
    afhi = afhi_ref[...]
    s = (jax.lax.dot_general(tfhi, afhi, dn,
                             preferred_element_type=jnp.float32)
         + jax.lax.dot_general(tfhi, aflo_ref[...], dn,
                               preferred_element_type=jnp.float32)
         + jax.lax.dot_general(tflo_ref[...], afhi, dn,
                               preferred_element_type=jnp.float32))
    s = s * (1.0 / jnp.sqrt(jnp.float32(d)))
    sim_ref[...] = s
    rmax = jnp.max(s, axis=1, keepdims=True)
    rmax_ref[...] = rmax
    e = jnp.exp(s - rmax)
    rsum_ref[...] = jnp.sum(e, axis=1, keepdims=True)
    cmax = jnp.max(s, axis=0)
    cmax_ref[0, 0, :] = cmax
    csum_ref[0, 0, :] = jnp.sum(jnp.exp(s - cmax[None, :]), axis=0)


def _sim_stats(tf_hilo, af_hilo):
    tfhi, tflo = tf_hilo
    afhi, aflo = af_hilo
    n, d = tfhi.shape
    m = afhi.shape[0]
    g = n // _RB
    return pl.pallas_call(
        _sim_stats_kernel,
        grid=(g,),
        in_specs=[
            pl.BlockSpec((_RB, d), lambda i: (i, 0)),
            pl.BlockSpec((_RB, d), lambda i: (i, 0)),
            pl.BlockSpec((m, d), lambda i: (0, 0)),
            pl.BlockSpec((m, d), lambda i: (0, 0)),
        ],
        out_specs=[
            pl.BlockSpec((_RB, m), lambda i: (i, 0)),
            pl.BlockSpec((_RB, 1), lambda i: (i, 0)),
            pl.BlockSpec((_RB, 1), lambda i: (i, 0)),
            pl.BlockSpec((1, 1, m), lambda i: (i, 0, 0)),
            pl.BlockSpec((1, 1, m), lambda i: (i, 0, 0)),
        ],
        out_shape=[
            jax.ShapeDtypeStruct((n, m), jnp.float32),
            jax.ShapeDtypeStruct((n, 1), jnp.float32),
            jax.ShapeDtypeStruct((n, 1), jnp.float32),
            jax.ShapeDtypeStruct((g, 1, m), jnp.float32),
            jax.ShapeDtypeStruct((g, 1, m), jnp.float32),
        ],
    )(tfhi, tflo, afhi, aflo)


def _dual_softmax_kernel(sim_ref, rmax_ref, rsum_ref, cmaxp_ref, csump_ref,
                         out_ref):
    s = sim_ref[...]
    g = cmaxp_ref.shape[0]
    m = cmaxp_ref.shape[2]
    cmaxp = cmaxp_ref[...].reshape(g, m)
    csump = csump_ref[...].reshape(g, m)
    cmax = jnp.max(cmaxp, axis=0)
    csum = jnp.sum(csump * jnp.exp(cmaxp - cmax[None, :]), axis=0)
    num = jnp.exp((s - rmax_ref[...]) + (s - cmax[None, :]))
    out_ref[...] = num / (rsum_ref[...] * csum[None, :])


def _dual_softmax(sim, rmax, rsum, cmaxp, csump):
    n, m = sim.shape
    g = cmaxp.shape[0]
    return pl.pallas_call(
        _dual_softmax_kernel,
        grid=(n // _RB,),
        in_specs=[
            pl.BlockSpec((_RB, m), lambda i: (i, 0)),
            pl.BlockSpec((_RB, 1), lambda i: (i, 0)),
            pl.BlockSpec((_RB, 1), lambda i: (i, 0)),
            pl.BlockSpec((g, 1, m), lambda i: (0, 0, 0)),
            pl.BlockSpec((g, 1, m), lambda i: (0, 0, 0)),
        ],
        out_specs=pl.BlockSpec((_RB, m), lambda i: (i, 0)),
        out_shape=jax.ShapeDtypeStruct((n, m), jnp.float32),
    )(sim, rmax, rsum, cmaxp, csump)


def kernel(anchor_coord, anchor_feat, anchor_offset, target_coord,
           target_feat, target_offset, Wq1, Wc1, Wq2, Wc2):
    tc8 = jnp.pad(target_coord, ((0, 0), (0, 5)))
    ac8 = jnp.pad(anchor_coord, ((0, 0), (0, 5)))

    dtt = _dist(tc8, tc8)
    dta = _dist(tc8, ac8)
    dat = _dist(ac8, tc8)
    t_tt = _sc_select(dtt, _K_QQ).reshape(-1, 1)
    t_ta = _sc_select(dta, _K_QC).reshape(-1, 1)
    t_at = _sc_select(dat, _K_QC).reshape(-1, 1)

    tf = target_feat
    af = anchor_feat
    tf_hl = _split(tf)
    af_hl = _split(af)
    for (wq, wc) in ((Wq1, Wc1), (Wq2, Wc2)):
        tf = _attn_step(dtt, t_tt, tf_hl, tf, wq)
        tf = _attn_step(dta, t_ta, af_hl, tf, wc)
        tf_hl = _split(tf)
        af = _attn_step(dat, t_at, tf_hl, af, wc)
        af_hl = _split(af)

    sim, rmax, rsum, cmaxp, csump = _sim_stats(tf_hl, af_hl)
    return _dual_softmax(sim, rmax, rsum, cmaxp, csump)
